# baseline (speedup 1.0000x reference)
import functools

import jax
import jax.numpy as jnp
from jax import lax
from jax.experimental import pallas as pl
from jax.experimental.pallas import tpu as pltpu
from jax.experimental.pallas import tpu_sc as plsc

_N = 10000
_E = 320000
_NSUB = 16
_CHUNK = 128
_NPAD = 10112
_RPS = _NPAD // _NSUB
_EPAD = 327680
_CPW = _EPAD // (32 * _CHUNK)
_CPS = _EPAD // (_NSUB * _CHUNK)
_BLK = 40


def _mesh():
    return plsc.VectorSubcoreMesh(core_axis_name="c", subcore_axis_name="s")


def _make_agg(esplit):
    out_type = [
        jax.ShapeDtypeStruct((_NPAD, 128), jnp.float32),
        jax.ShapeDtypeStruct((_NPAD, 128), jnp.float32),
    ]
    scratch = [
        pltpu.VMEM((_BLK, 2, _CHUNK), jnp.int32),
        pltpu.VMEM((_CHUNK, 128), jnp.float32),
        pltpu.VMEM((_CHUNK, 128), jnp.float32),
        pltpu.VMEM_SHARED((_NPAD, 128), jnp.float32),
        pltpu.SemaphoreType.DMA,
        pltpu.SemaphoreType.DMA,
    ]

    @functools.partial(
        pl.kernel, mesh=_mesh(), out_type=out_type, scratch_types=scratch
    )
    def k(e_r, hlo_r, hhi_r, zf_r, alo_r, ahi_r,
          idxb, rows0, rows1, agg_sh, sem0, sem1):
        rows = (rows0, rows1)
        sem = (sem0, sem1)
        cid = lax.axis_index("c")
        sid = lax.axis_index("s")
        r0 = sid * _RPS
        pltpu.sync_copy(zf_r.at[pl.ds(r0, _RPS)], agg_sh.at[pl.ds(r0, _RPS)])
        plsc.subcore_barrier()

        if esplit:
            nblocks = _CPW // _BLK
            cbase = (cid * _NSUB + sid) * _CPW
        else:
            nblocks = _CPS // _BLK
            cbase = sid * _CPS

        def issue_gather(i, b):
            if esplit:
                pltpu.async_copy(hlo_r.at[idxb.at[i, 0]], rows[b], sem[b])
            else:
                @pl.when(cid == 0)
                def _():
                    pltpu.async_copy(hlo_r.at[idxb.at[i, 0]], rows[b],
                                     sem[b])

                @pl.when(cid == 1)
                def _():
                    pltpu.async_copy(hhi_r.at[idxb.at[i, 0]], rows[b],
                                     sem[b])

        def wait_gather(i, b):
            pltpu.make_async_copy(hlo_r.at[idxb.at[i, 0]], rows[b],
                                  sem[b]).wait()

        def pair(g, carry):
            for b in range(2):
                i = g * 2 + b
                wait_gather(i, b)
                if b == 0:
                    issue_gather(i + 1, 1)
                else:
                    @pl.when(g < _BLK // 2 - 1)
                    def _():
                        issue_gather(i + 1, 0)
                pltpu.sync_copy(rows[b], agg_sh.at[idxb.at[i, 1]], add=True)
            return carry

        for blk in range(nblocks):
            pltpu.sync_copy(e_r.at[pl.ds(cbase + blk * _BLK, _BLK)], idxb)
            issue_gather(0, 0)
            lax.fori_loop(0, _BLK // 2, pair, 0)
        plsc.subcore_barrier()

        @pl.when(cid == 0)
        def _():
            pltpu.sync_copy(agg_sh.at[pl.ds(r0, _RPS)],
                            alo_r.at[pl.ds(r0, _RPS)])

        @pl.when(cid == 1)
        def _():
            pltpu.sync_copy(agg_sh.at[pl.ds(r0, _RPS)],
                            ahi_r.at[pl.ds(r0, _RPS)])

    return k


def _make_cnt():
    out_type = [
        jax.ShapeDtypeStruct((_NPAD, 128), jnp.float32),
        jax.ShapeDtypeStruct((_NPAD, 128), jnp.float32),
    ]
    scratch = [
        pltpu.VMEM((_CHUNK,), jnp.int32),
        pltpu.VMEM((_CHUNK, 128), jnp.float32),
        pltpu.VMEM_SHARED((_NPAD, 128), jnp.float32),
    ]

    @functools.partial(
        pl.kernel, mesh=_mesh(), out_type=out_type, scratch_types=scratch
    )
    def k(dst_r, zf_r, ones_r, c0_r, c1_r, didx, ones_v, cnt_sh):
        cid = lax.axis_index("c")
        sid = lax.axis_index("s")
        r0 = sid * _RPS
        pltpu.sync_copy(zf_r.at[pl.ds(r0, _RPS)], cnt_sh.at[pl.ds(r0, _RPS)])
        pltpu.sync_copy(ones_r, ones_v)
        plsc.subcore_barrier()

        cbase = (cid * _NSUB + sid) * _CPW

        def step(i, carry):
            base = pl.multiple_of((cbase + i) * _CHUNK, _CHUNK)
            pltpu.sync_copy(dst_r.at[pl.ds(base, _CHUNK)], didx)
            pltpu.sync_copy(ones_v, cnt_sh.at[didx], add=True)
            return carry

        lax.fori_loop(0, _CPW, step, 0)
        plsc.subcore_barrier()

        @pl.when(cid == 0)
        def _():
            pltpu.sync_copy(cnt_sh.at[pl.ds(r0, _RPS)],
                            c0_r.at[pl.ds(r0, _RPS)])

        @pl.when(cid == 1)
        def _():
            pltpu.sync_copy(cnt_sh.at[pl.ds(r0, _RPS)],
                            c1_r.at[pl.ds(r0, _RPS)])

    return k


_agg_l0 = _make_agg(esplit=True)
_agg128 = _make_agg(esplit=False)
_cnt_kernel = _make_cnt()


_BN = 1000


def _full(shape):
    return pl.BlockSpec(shape, lambda i: (0,) * len(shape))


def _rows(shape):
    return pl.BlockSpec(shape, lambda i: (i,) + (0,) * (len(shape) - 1))


def _proj0_kern(x_r, w_r, b_r, h_r):
    h = jnp.dot(x_r[...], w_r[...], preferred_element_type=jnp.float32)
    h_r[...] = jnp.maximum(h + b_r[...], 0.0)


def _proj0(x, pw, pb):
    return pl.pallas_call(
        _proj0_kern,
        grid=(_N // _BN,),
        in_specs=[_rows((_BN, 128)), _full((128, 128)), _full((1, 128))],
        out_specs=_rows((_BN, 128)),
        out_shape=jax.ShapeDtypeStruct((_N, 128), jnp.float32),
    )(x, pw, pb.reshape(1, -1))


def _comb_kern(split, alo_r, ahi_r, c0_r, c1_r, xp_r, lw_r, lb_r, rw_r,
               pw_r, pb_r, xo_r, lo_r, hi_r):
    cnt = jnp.maximum(c0_r[:, :1] + c1_r[:, :1], 1.0)
    inv = 1.0 / cnt
    lw = lw_r[...]
    if split:
        out = jnp.dot(alo_r[...] * inv, lw[:128],
                      preferred_element_type=jnp.float32)
        out += jnp.dot(ahi_r[...] * inv, lw[128:],
                       preferred_element_type=jnp.float32)
    else:
        out = jnp.dot((alo_r[...] + ahi_r[...]) * inv, lw,
                      preferred_element_type=jnp.float32)
    out += jnp.dot(xp_r[...], rw_r[...], preferred_element_type=jnp.float32)
    out += lb_r[...]
    nrm = jnp.sqrt(jnp.sum(out * out, axis=1, keepdims=True))
    out = out / jnp.maximum(nrm, 1e-12)
    x1 = out * jax.nn.sigmoid(out)
    xo_r[...] = x1
    h = jnp.dot(x1, pw_r[...], preferred_element_type=jnp.float32)
    h = jnp.maximum(h + pb_r[...], 0.0)
    lo_r[...] = h[:, :128]
    hi_r[...] = h[:, 128:]


def _combine(split, ic, alo, ahi, c0, c1, xp, lw, lb, rw, pw, pb):
    return pl.pallas_call(
        functools.partial(_comb_kern, split),
        grid=(_N // _BN,),
        in_specs=[
            _rows((_BN, 128)), _rows((_BN, 128)),
            _rows((_BN, 16)), _rows((_BN, 16)),
            _rows((_BN, ic)), _full((ic, 256)), _full((1, 256)),
            _full((ic, 256)), _full((256, 256)), _full((1, 256)),
        ],
        out_specs=[_rows((_BN, 256)), _rows((_BN, 128)), _rows((_BN, 128))],
        out_shape=[
            jax.ShapeDtypeStruct((_N, 256), jnp.float32),
            jax.ShapeDtypeStruct((_N, 128), jnp.float32),
            jax.ShapeDtypeStruct((_N, 128), jnp.float32),
        ],
    )(alo, ahi, c0, c1, xp, lw, lb.reshape(1, -1), rw, pw, pb.reshape(1, -1))


def _final_kern(alo_r, ahi_r, c0_r, c1_r, xp_r, lw_r, lb_r, rw_r,
                g_r, bt_r, o_r):
    cnt = jnp.maximum(c0_r[:, :1] + c1_r[:, :1], 1.0)
    inv = 1.0 / cnt
    lw = lw_r[...]
    out = jnp.dot(alo_r[...] * inv, lw[:128],
                  preferred_element_type=jnp.float32)
    out += jnp.dot(ahi_r[...] * inv, lw[128:],
                   preferred_element_type=jnp.float32)
    out += jnp.dot(xp_r[...], rw_r[...], preferred_element_type=jnp.float32)
    out += lb_r[...]
    nrm = jnp.sqrt(jnp.sum(out * out, axis=1, keepdims=True))
    h = out / jnp.maximum(nrm, 1e-12)
    mu = jnp.mean(h, axis=1, keepdims=True)
    d = h - mu
    var = jnp.mean(d * d, axis=1, keepdims=True)
    o_r[...] = d / jnp.sqrt(var + 1e-5) * g_r[...] + bt_r[...]


def _final(alo, ahi, c0, c1, xp, lw, lb, rw, g, b):
    return pl.pallas_call(
        _final_kern,
        grid=(_N // _BN,),
        in_specs=[
            _rows((_BN, 128)), _rows((_BN, 128)),
            _rows((_BN, 16)), _rows((_BN, 16)),
            _rows((_BN, 256)), _full((256, 256)), _full((1, 256)),
            _full((256, 256)), _full((1, 256)), _full((1, 256)),
        ],
        out_specs=_rows((_BN, 256)),
        out_shape=jax.ShapeDtypeStruct((_N, 256), jnp.float32),
    )(alo, ahi, c0, c1, xp, lw, lb.reshape(1, -1), rw,
      g.reshape(1, -1), b.reshape(1, -1))


def kernel(x, edge_index, proj_W0, proj_b0, linl_W0, linl_b0, linr_W0,
           proj_W1, proj_b1, linl_W1, linl_b1, linr_W1,
           proj_W2, proj_b2, linl_W2, linl_b2, linr_W2,
           ln_gamma, ln_beta):
    src = edge_index[0]
    dst = edge_index[1]
    pad = _EPAD - _E
    src_p = jnp.concatenate([src, jnp.zeros((pad,), jnp.int32)])
    dst_p = jnp.concatenate([dst, jnp.full((pad,), _N, jnp.int32)])
    e_p = jnp.stack([src_p.reshape(-1, _CHUNK),
                     dst_p.reshape(-1, _CHUNK)], axis=1)
    z128 = jnp.zeros((_NPAD, 128), jnp.float32)
    ones128 = jnp.ones((_CHUNK, 128), jnp.float32)

    c0f, c1f = _cnt_kernel(dst_p, z128, ones128)
    c0 = c0f[:_N, :16]
    c1 = c1f[:_N, :16]
    h0 = _proj0(x, proj_W0, proj_b0)
    alo, ahi = _agg_l0(e_p, h0, h0, z128)
    x1, h_lo, h_hi = _combine(False, 128, alo[:_N], ahi[:_N], c0, c1, x,
                              linl_W0, linl_b0, linr_W0, proj_W1, proj_b1)
    alo, ahi = _agg128(e_p, h_lo, h_hi, z128)
    x2, h_lo, h_hi = _combine(True, 256, alo[:_N], ahi[:_N], c0, c1, x1,
                              linl_W1, linl_b1, linr_W1, proj_W2, proj_b2)
    alo, ahi = _agg128(e_p, h_lo, h_hi, z128)
    return _final(alo[:_N], ahi[:_N], c0, c1, x2,
                  linl_W2, linl_b2, linr_W2, ln_gamma, ln_beta)


# speedup vs baseline: 1.3044x; 1.3044x over previous
import functools

import jax
import jax.numpy as jnp
from jax import lax
from jax.experimental import pallas as pl
from jax.experimental.pallas import tpu as pltpu
from jax.experimental.pallas import tpu_sc as plsc

_N = 10000
_E = 320000
_NSUB = 16
_CHUNK = 128
_NPAD = 10112
_RPS = _NPAD // _NSUB
_EPAD = 323584
_CPW = _EPAD // (32 * _CHUNK)
_CPS = _EPAD // (_NSUB * _CHUNK)


def _mesh():
    return plsc.VectorSubcoreMesh(core_axis_name="c", subcore_axis_name="s")


def _make_agg(esplit):
    out_type = [
        jax.ShapeDtypeStruct((_NPAD, 128), jnp.float32),
        jax.ShapeDtypeStruct((_NPAD, 128), jnp.float32),
    ]
    scratch = [
        pltpu.VMEM((_CPW, 2, _CHUNK), jnp.int32),
        pltpu.VMEM((_CHUNK, 128), jnp.float32),
        pltpu.VMEM_SHARED((_NPAD, 128), jnp.float32),
        pltpu.SemaphoreType.DMA,
    ]

    @functools.partial(
        pl.kernel, mesh=_mesh(), out_type=out_type, scratch_types=scratch
    )
    def k(e_r, hlo_r, hhi_r, zf_r, alo_r, ahi_r,
          idxb, rows, agg_sh, sem):
        cid = lax.axis_index("c")
        sid = lax.axis_index("s")
        r0 = sid * _RPS
        pltpu.sync_copy(zf_r.at[pl.ds(r0, _RPS)], agg_sh.at[pl.ds(r0, _RPS)])
        plsc.subcore_barrier()

        if esplit:
            nblocks = 1
            cbase = (cid * _NSUB + sid) * _CPW
        else:
            nblocks = 2
            cbase = sid * _CPS

        def step(i, carry):
            if esplit:
                pltpu.async_copy(hlo_r.at[idxb.at[i, 0]], rows, sem).wait()
            else:
                @pl.when(cid == 0)
                def _():
                    pltpu.async_copy(hlo_r.at[idxb.at[i, 0]], rows,
                                     sem).wait()

                @pl.when(cid == 1)
                def _():
                    pltpu.async_copy(hhi_r.at[idxb.at[i, 0]], rows,
                                     sem).wait()

            pltpu.sync_copy(rows, agg_sh.at[idxb.at[i, 1]], add=True)
            return carry

        for blk in range(nblocks):
            pltpu.sync_copy(e_r.at[pl.ds(cbase + blk * _CPW, _CPW)], idxb)
            lax.fori_loop(0, _CPW, step, 0)
        plsc.subcore_barrier()

        @pl.when(cid == 0)
        def _():
            pltpu.sync_copy(agg_sh.at[pl.ds(r0, _RPS)],
                            alo_r.at[pl.ds(r0, _RPS)])

        @pl.when(cid == 1)
        def _():
            pltpu.sync_copy(agg_sh.at[pl.ds(r0, _RPS)],
                            ahi_r.at[pl.ds(r0, _RPS)])

    return k


def _make_cnt():
    out_type = [
        jax.ShapeDtypeStruct((_NPAD, 128), jnp.float32),
        jax.ShapeDtypeStruct((_NPAD, 128), jnp.float32),
    ]
    scratch = [
        pltpu.VMEM((_CHUNK,), jnp.int32),
        pltpu.VMEM((_CHUNK, 128), jnp.float32),
        pltpu.VMEM_SHARED((_NPAD, 128), jnp.float32),
    ]

    @functools.partial(
        pl.kernel, mesh=_mesh(), out_type=out_type, scratch_types=scratch
    )
    def k(dst_r, zf_r, ones_r, c0_r, c1_r, didx, ones_v, cnt_sh):
        cid = lax.axis_index("c")
        sid = lax.axis_index("s")
        r0 = sid * _RPS
        pltpu.sync_copy(zf_r.at[pl.ds(r0, _RPS)], cnt_sh.at[pl.ds(r0, _RPS)])
        pltpu.sync_copy(ones_r, ones_v)
        plsc.subcore_barrier()

        cbase = (cid * _NSUB + sid) * _CPW

        def step(i, carry):
            base = pl.multiple_of((cbase + i) * _CHUNK, _CHUNK)
            pltpu.sync_copy(dst_r.at[pl.ds(base, _CHUNK)], didx)
            pltpu.sync_copy(ones_v, cnt_sh.at[didx], add=True)
            return carry

        lax.fori_loop(0, _CPW, step, 0)
        plsc.subcore_barrier()

        @pl.when(cid == 0)
        def _():
            pltpu.sync_copy(cnt_sh.at[pl.ds(r0, _RPS)],
                            c0_r.at[pl.ds(r0, _RPS)])

        @pl.when(cid == 1)
        def _():
            pltpu.sync_copy(cnt_sh.at[pl.ds(r0, _RPS)],
                            c1_r.at[pl.ds(r0, _RPS)])

    return k


_agg_l0 = _make_agg(esplit=True)
_agg128 = _make_agg(esplit=False)
_cnt_kernel = _make_cnt()


_BN = 1000


def _full(shape):
    return pl.BlockSpec(shape, lambda i: (0,) * len(shape))


def _rows(shape):
    return pl.BlockSpec(shape, lambda i: (i,) + (0,) * (len(shape) - 1))


def _proj0_kern(x_r, w_r, b_r, h_r):
    h = jnp.dot(x_r[...], w_r[...], preferred_element_type=jnp.float32)
    h_r[...] = jnp.maximum(h + b_r[...], 0.0)


def _proj0(x, pw, pb):
    return pl.pallas_call(
        _proj0_kern,
        grid=(_N // _BN,),
        in_specs=[_rows((_BN, 128)), _full((128, 128)), _full((1, 128))],
        out_specs=_rows((_BN, 128)),
        out_shape=jax.ShapeDtypeStruct((_N, 128), jnp.float32),
    )(x, pw, pb.reshape(1, -1))


def _comb_kern(split, alo_r, ahi_r, c0_r, c1_r, xp_r, lw_r, lb_r, rw_r,
               pw_r, pb_r, xo_r, lo_r, hi_r):
    cnt = jnp.maximum(c0_r[:, :1] + c1_r[:, :1], 1.0)
    inv = 1.0 / cnt
    lw = lw_r[...]
    if split:
        out = jnp.dot(alo_r[...] * inv, lw[:128],
                      preferred_element_type=jnp.float32)
        out += jnp.dot(ahi_r[...] * inv, lw[128:],
                       preferred_element_type=jnp.float32)
    else:
        out = jnp.dot((alo_r[...] + ahi_r[...]) * inv, lw,
                      preferred_element_type=jnp.float32)
    out += jnp.dot(xp_r[...], rw_r[...], preferred_element_type=jnp.float32)
    out += lb_r[...]
    nrm = jnp.sqrt(jnp.sum(out * out, axis=1, keepdims=True))
    out = out / jnp.maximum(nrm, 1e-12)
    x1 = out * jax.nn.sigmoid(out)
    xo_r[...] = x1
    h = jnp.dot(x1, pw_r[...], preferred_element_type=jnp.float32)
    h = jnp.maximum(h + pb_r[...], 0.0)
    lo_r[...] = h[:, :128]
    hi_r[...] = h[:, 128:]


def _combine(split, ic, alo, ahi, c0, c1, xp, lw, lb, rw, pw, pb):
    return pl.pallas_call(
        functools.partial(_comb_kern, split),
        grid=(_N // _BN,),
        in_specs=[
            _rows((_BN, 128)), _rows((_BN, 128)),
            _rows((_BN, 16)), _rows((_BN, 16)),
            _rows((_BN, ic)), _full((ic, 256)), _full((1, 256)),
            _full((ic, 256)), _full((256, 256)), _full((1, 256)),
        ],
        out_specs=[_rows((_BN, 256)), _rows((_BN, 128)), _rows((_BN, 128))],
        out_shape=[
            jax.ShapeDtypeStruct((_N, 256), jnp.float32),
            jax.ShapeDtypeStruct((_N, 128), jnp.float32),
            jax.ShapeDtypeStruct((_N, 128), jnp.float32),
        ],
    )(alo, ahi, c0, c1, xp, lw, lb.reshape(1, -1), rw, pw, pb.reshape(1, -1))


def _final_kern(alo_r, ahi_r, c0_r, c1_r, xp_r, lw_r, lb_r, rw_r,
                g_r, bt_r, o_r):
    cnt = jnp.maximum(c0_r[:, :1] + c1_r[:, :1], 1.0)
    inv = 1.0 / cnt
    lw = lw_r[...]
    out = jnp.dot(alo_r[...] * inv, lw[:128],
                  preferred_element_type=jnp.float32)
    out += jnp.dot(ahi_r[...] * inv, lw[128:],
                   preferred_element_type=jnp.float32)
    out += jnp.dot(xp_r[...], rw_r[...], preferred_element_type=jnp.float32)
    out += lb_r[...]
    nrm = jnp.sqrt(jnp.sum(out * out, axis=1, keepdims=True))
    h = out / jnp.maximum(nrm, 1e-12)
    mu = jnp.mean(h, axis=1, keepdims=True)
    d = h - mu
    var = jnp.mean(d * d, axis=1, keepdims=True)
    o_r[...] = d / jnp.sqrt(var + 1e-5) * g_r[...] + bt_r[...]


def _final(alo, ahi, c0, c1, xp, lw, lb, rw, g, b):
    return pl.pallas_call(
        _final_kern,
        grid=(_N // _BN,),
        in_specs=[
            _rows((_BN, 128)), _rows((_BN, 128)),
            _rows((_BN, 16)), _rows((_BN, 16)),
            _rows((_BN, 256)), _full((256, 256)), _full((1, 256)),
            _full((256, 256)), _full((1, 256)), _full((1, 256)),
        ],
        out_specs=_rows((_BN, 256)),
        out_shape=jax.ShapeDtypeStruct((_N, 256), jnp.float32),
    )(alo, ahi, c0, c1, xp, lw, lb.reshape(1, -1), rw,
      g.reshape(1, -1), b.reshape(1, -1))


def kernel(x, edge_index, proj_W0, proj_b0, linl_W0, linl_b0, linr_W0,
           proj_W1, proj_b1, linl_W1, linl_b1, linr_W1,
           proj_W2, proj_b2, linl_W2, linl_b2, linr_W2,
           ln_gamma, ln_beta):
    src = edge_index[0]
    dst = edge_index[1]
    pad = _EPAD - _E
    src_p = jnp.concatenate([src, jnp.zeros((pad,), jnp.int32)])
    dst_p = jnp.concatenate([dst, jnp.full((pad,), _N, jnp.int32)])
    e_p = jnp.stack([src_p.reshape(-1, _CHUNK),
                     dst_p.reshape(-1, _CHUNK)], axis=1)
    z128 = jnp.zeros((_NPAD, 128), jnp.float32)
    ones128 = jnp.ones((_CHUNK, 128), jnp.float32)

    c0f, c1f = _cnt_kernel(dst_p, z128, ones128)
    c0 = c0f[:_N, :16]
    c1 = c1f[:_N, :16]
    h0 = _proj0(x, proj_W0, proj_b0)
    alo, ahi = _agg_l0(e_p, h0, h0, z128)
    x1, h_lo, h_hi = _combine(False, 128, alo[:_N], ahi[:_N], c0, c1, x,
                              linl_W0, linl_b0, linr_W0, proj_W1, proj_b1)
    alo, ahi = _agg128(e_p, h_lo, h_hi, z128)
    x2, h_lo, h_hi = _combine(True, 256, alo[:_N], ahi[:_N], c0, c1, x1,
                              linl_W1, linl_b1, linr_W1, proj_W2, proj_b2)
    alo, ahi = _agg128(e_p, h_lo, h_hi, z128)
    return _final(alo[:_N], ahi[:_N], c0, c1, x2,
                  linl_W2, linl_b2, linr_W2, ln_gamma, ln_beta)
